# Initial kernel scaffold; baseline (speedup 1.0000x reference)
import jax
import jax.numpy as jnp
from jax.experimental import pallas as pl


def kernel(x, edge_index, m0c0W1, m0c0b1, m0c0W2, m0c0b2, m0c1W1, m0c1b1, m0c1W2, m0c1b2, m1c0W1, m1c0b1, m1c0W2, m1c0b2, m1c1W1, m1c1b1, m1c1W2, m1c1b2, Wout, bout):
    raise NotImplementedError("write your pallas kernel here")


# speedup vs baseline: 11.1482x; 11.1482x over previous
import functools

import jax
import jax.numpy as jnp
from jax import lax
from jax.experimental import pallas as pl
from jax.experimental.pallas import tpu as pltpu
from jax.experimental.pallas import tpu_sc as plsc

N = 10000
E = 320000
D = 128

NC = 2
NS = 16
NW = NC * NS
EPW = E // NW
CH = 128
NFULL = EPW // CH
TAIL = EPW - NFULL * CH
NPAD = 10240
RPS = NPAD // NS
ZCH = 128


def _sc_agg_body(h_hbm, src_hbm, dst_hbm, out_hbm,
                 sv0, sv1, sv2, sv3, dv0, dv1, dv2, dv3,
                 rows0, rows1, st, dt, rt,
                 si0, si1, si2, si3, di0, di1, di2, di3,
                 sg0, sg1, ss0, ss1, agg_sh):
    sv = [sv0, sv1, sv2, sv3]
    dv = [dv0, dv1, dv2, dv3]
    rows = [rows0, rows1]
    si = [si0, si1, si2, si3]
    di = [di0, di1, di2, di3]
    sg = [sg0, sg1]
    ss = [ss0, ss1]

    cid = lax.axis_index("c")
    sid = lax.axis_index("s")
    wid = sid * NC + cid
    base = wid * EPW

    zero16 = jnp.zeros((16,), jnp.float32)

    @pl.loop(0, ZCH)
    def _(r):
        @pl.loop(0, D, step=16)
        def _(c):
            rows0[r, pl.ds(c, 16)] = zero16

    for k in range(5):
        r0 = sid * RPS + k * ZCH
        pltpu.sync_copy(rows0, agg_sh.at[pl.ds(r0, ZCH)])

    plsc.subcore_barrier()

    def start_idx(c, s4):
        pltpu.async_copy(src_hbm.at[pl.ds(base + c * CH, CH)], sv[s4], si[s4])
        pltpu.async_copy(dst_hbm.at[pl.ds(base + c * CH, CH)], dv[s4], di[s4])

    def wait_idx(c, s4):
        pltpu.make_async_copy(src_hbm.at[pl.ds(base + c * CH, CH)], sv[s4], si[s4]).wait()
        pltpu.make_async_copy(dst_hbm.at[pl.ds(base + c * CH, CH)], dv[s4], di[s4]).wait()

    def start_gather(s4, s2):
        pltpu.async_copy(h_hbm.at[sv[s4]], rows[s2], sg[s2])

    def wait_gather(s4, s2):
        pltpu.make_async_copy(h_hbm.at[sv[s4]], rows[s2], sg[s2]).wait()

    def start_scatter(s4, s2):
        pltpu.async_copy(rows[s2], agg_sh.at[dv[s4]], ss[s2], add=True)

    def wait_scatter(s4, s2):
        pltpu.make_async_copy(rows[s2], agg_sh.at[dv[s4]], ss[s2]).wait()

    def chunk_step(c, b, first):
        s4, s2 = b % 4, b % 2
        if not first:
            wait_scatter((b + 2) % 4, s2)
        start_idx(c + 2, (b + 2) % 4)
        wait_idx(c, s4)
        start_gather(s4, s2)
        wait_gather(s4, s2)
        start_scatter(s4, s2)

    start_idx(0, 0)
    start_idx(1, 1)
    for b in range(4):
        chunk_step(b, b, first=(b < 2))

    @pl.loop(4, 76, step=4)
    def _(c0):
        for b in range(4):
            chunk_step(c0 + b, b, first=False)

    for c in (76, 77):
        s4, s2 = c % 4, c % 2
        wait_scatter((s4 + 2) % 4, s2)
        wait_idx(c, s4)
        start_gather(s4, s2)
        wait_gather(s4, s2)
        start_scatter(s4, s2)
    wait_scatter(76 % 4, 0)
    wait_scatter(77 % 4, 1)

    toff = base + NFULL * CH
    pltpu.sync_copy(src_hbm.at[pl.ds(toff, TAIL)], st)
    pltpu.sync_copy(dst_hbm.at[pl.ds(toff, TAIL)], dt)
    pltpu.async_copy(h_hbm.at[st], rt, sg0).wait()
    pltpu.sync_copy(rt, agg_sh.at[dt], add=True)

    plsc.subcore_barrier()

    for k in range(5):
        r0 = sid * RPS + k * ZCH
        pltpu.sync_copy(agg_sh.at[pl.ds(r0, ZCH)], out_hbm.at[cid, pl.ds(r0, ZCH)])


@functools.cache
def _get_sc_aggregate():
    mesh = plsc.VectorSubcoreMesh(core_axis_name="c", subcore_axis_name="s",
                                  num_cores=NC, num_subcores=NS)
    return pl.kernel(
        _sc_agg_body,
        out_type=jax.ShapeDtypeStruct((NC, NPAD, D), jnp.float32),
        mesh=mesh,
        scratch_types=(
            [pltpu.VMEM((CH,), jnp.int32) for _ in range(8)]
            + [pltpu.VMEM((CH, D), jnp.float32) for _ in range(2)]
            + [pltpu.VMEM((TAIL,), jnp.int32) for _ in range(2)]
            + [pltpu.VMEM((TAIL, D), jnp.float32)]
            + [pltpu.SemaphoreType.DMA for _ in range(12)]
            + [pltpu.VMEM_SHARED((NPAD, D), jnp.float32)]
        ),
    )


def _sc_aggregate(h, src, dst):
    return _get_sc_aggregate()(h, src, dst)


def _mlp_block(h_ref, agg_ref, w1_ref, b1_ref, w2_ref, b2_ref, o_ref, *, relu_out):
    z = h_ref[...] + agg_ref[0] + agg_ref[1]
    t = jnp.dot(z, w1_ref[...], preferred_element_type=jnp.float32) + b1_ref[...]
    t = jnp.maximum(t, 0.0)
    o = jnp.dot(t, w2_ref[...], preferred_element_type=jnp.float32) + b2_ref[...]
    if relu_out:
        o = jnp.maximum(o, 0.0)
    o_ref[...] = o


def _mlp_final_block(h_ref, agg_ref, w1_ref, b1_ref, w2_ref, b2_ref,
                     wo_ref, bo_ref, o_ref):
    z = h_ref[...] + agg_ref[0] + agg_ref[1]
    t = jnp.dot(z, w1_ref[...], preferred_element_type=jnp.float32) + b1_ref[...]
    t = jnp.maximum(t, 0.0)
    o = jnp.dot(t, w2_ref[...], preferred_element_type=jnp.float32) + b2_ref[...]
    o_ref[...] = jnp.dot(o, wo_ref[...], preferred_element_type=jnp.float32) + bo_ref[...]


_BM = 1000

_row_spec = pl.BlockSpec((_BM, D), lambda i: (i, 0))
_agg_spec = pl.BlockSpec((NC, _BM, D), lambda i: (0, i, 0))
_w_spec = pl.BlockSpec((D, D), lambda i: (0, 0))
_b_spec = pl.BlockSpec((1, D), lambda i: (0, 0))


def _tc_mlp(h, agg, w1, b1, w2, b2, relu_out):
    return pl.pallas_call(
        functools.partial(_mlp_block, relu_out=relu_out),
        grid=(N // _BM,),
        in_specs=[_row_spec, _agg_spec, _w_spec, _b_spec, _w_spec, _b_spec],
        out_specs=_row_spec,
        out_shape=jax.ShapeDtypeStruct((N, D), jnp.float32),
    )(h, agg, w1, b1.reshape(1, D), w2, b2.reshape(1, D))


def _tc_mlp_final(h, agg, w1, b1, w2, b2, wo, bo):
    return pl.pallas_call(
        _mlp_final_block,
        grid=(N // _BM,),
        in_specs=[_row_spec, _agg_spec, _w_spec, _b_spec, _w_spec, _b_spec,
                  _w_spec, _b_spec],
        out_specs=_row_spec,
        out_shape=jax.ShapeDtypeStruct((N, D), jnp.float32),
    )(h, agg, w1, b1.reshape(1, D), w2, b2.reshape(1, D), wo, bo.reshape(1, D))


def kernel(x, edge_index,
           m0c0W1, m0c0b1, m0c0W2, m0c0b2,
           m0c1W1, m0c1b1, m0c1W2, m0c1b2,
           m1c0W1, m1c0b1, m1c0W2, m1c0b2,
           m1c1W1, m1c1b1, m1c1W2, m1c1b2,
           Wout, bout):
    src = edge_index[0]
    dst = edge_index[1]
    params = [
        (m0c0W1, m0c0b1, m0c0W2, m0c0b2),
        (m0c1W1, m0c1b1, m0c1W2, m0c1b2),
        (m1c0W1, m1c0b1, m1c0W2, m1c0b2),
        (m1c1W1, m1c1b1, m1c1W2, m1c1b2),
    ]
    h = x
    for l in range(3):
        w1, b1, w2, b2 = params[l]
        agg = _sc_aggregate(h, src, dst)
        h = _tc_mlp(h, agg, w1, b1, w2, b2, relu_out=(l in (0, 2)))
    w1, b1, w2, b2 = params[3]
    agg = _sc_aggregate(h, src, dst)
    return _tc_mlp_final(h, agg, w1, b1, w2, b2, Wout, bout)
